# baseline (speedup 1.0000x reference)
import functools

import jax
import jax.numpy as jnp
from jax import lax
from jax.experimental import pallas as pl
from jax.experimental.pallas import tpu as pltpu
from jax.experimental.pallas import tpu_sc as plsc

NC, NS = 2, 16
NW = NC * NS
CH = 16384
NBUF = 3


def _sc_copy_body(v_hbm, vo_hbm, buf, rs0, rs1, rs2, ws0, ws1, ws2):
    wid = lax.axis_index("s") * NC + lax.axis_index("c")
    n = v_hbm.shape[0]
    per_w = n // NW
    base = wid * per_w
    nchunks = per_w // CH
    rsems = (rs0, rs1, rs2)
    wsems = (ws0, ws1, ws2)
    offs = [c * CH for c in range(nchunks)]
    reads = [
        pltpu.make_async_copy(
            v_hbm.at[pl.ds(base + off, CH)], buf.at[i % NBUF], rsems[i % NBUF])
        for i, off in enumerate(offs)
    ]
    writes = [None] * nchunks
    for i in range(min(NBUF, nchunks)):
        reads[i].start()
    for i, off in enumerate(offs):
        reads[i].wait()
        w = pltpu.make_async_copy(
            buf.at[i % NBUF], vo_hbm.at[pl.ds(base + off, CH)], wsems[i % NBUF])
        w.start()
        writes[i] = w
        if i + NBUF < nchunks:
            writes[i].wait()
            reads[i + NBUF].start()
    for i in range(max(0, nchunks - NBUF), nchunks):
        writes[i].wait()


def _sc_copy(v_flat):
    n = v_flat.shape[0]
    mesh = plsc.VectorSubcoreMesh(core_axis_name="c", subcore_axis_name="s")
    fn = functools.partial(
        pl.kernel,
        mesh=mesh,
        out_type=jax.ShapeDtypeStruct((n,), jnp.float32),
        scratch_types=[
            pltpu.VMEM((NBUF, CH), jnp.float32),
            pltpu.SemaphoreType.DMA,
            pltpu.SemaphoreType.DMA,
            pltpu.SemaphoreType.DMA,
            pltpu.SemaphoreType.DMA,
            pltpu.SemaphoreType.DMA,
            pltpu.SemaphoreType.DMA,
        ],
    )(_sc_copy_body)
    return fn(v_flat)


def _tc_copy_kernel(k_ref, k_out_ref):
    k_out_ref[...] = k_ref[...]


def _tc_copy(k2):
    rows, cols = k2.shape
    blk = 256
    return pl.pallas_call(
        _tc_copy_kernel,
        grid=(rows // blk,),
        in_specs=[pl.BlockSpec((blk, cols), lambda i: (i, 0))],
        out_specs=pl.BlockSpec((blk, cols), lambda i: (i, 0)),
        out_shape=jax.ShapeDtypeStruct((rows, cols), k2.dtype),
    )(k2)


def kernel(k_val, v_val, k_cache, v_cache):
    del k_cache, v_cache
    b, s, h, d = k_val.shape
    n = b * s * h * d
    v_out = _sc_copy(v_val.reshape(n))
    k_out = _tc_copy(k_val.reshape(b * s, h * d))
    return (k_out.reshape(b, s, h, d), v_out.reshape(b, s, h, d))


# speedup vs baseline: 4.7960x; 4.7960x over previous
import jax
import jax.numpy as jnp
from jax.experimental import pallas as pl


def _copy_kernel(k_ref, v_ref, k_out_ref, v_out_ref):
    k_out_ref[...] = k_ref[...]
    v_out_ref[...] = v_ref[...]


def kernel(k_val, v_val, k_cache, v_cache):
    del k_cache, v_cache
    b, s, h, d = k_val.shape
    blk = 4
    spec = pl.BlockSpec((blk, s, h, d), lambda i: (i, 0, 0, 0))
    k_out, v_out = pl.pallas_call(
        _copy_kernel,
        grid=(b // blk,),
        in_specs=[spec, spec],
        out_specs=[spec, spec],
        out_shape=[
            jax.ShapeDtypeStruct((b, s, h, d), k_val.dtype),
            jax.ShapeDtypeStruct((b, s, h, d), v_val.dtype),
        ],
    )(k_val, v_val)
    return (k_out, v_out)


# speedup vs baseline: 5.7461x; 1.1981x over previous
import jax
import jax.numpy as jnp
from jax.experimental import pallas as pl


def _copy_kernel(k_ref, v_ref, k_out_ref, v_out_ref):
    k_out_ref[...] = k_ref[...]
    v_out_ref[...] = v_ref[...]


def kernel(k_val, v_val, k_cache, v_cache):
    del k_cache, v_cache
    b, s, h, d = k_val.shape
    blk = 8
    spec = pl.BlockSpec((blk, s, h, d), lambda i: (i, 0, 0, 0))
    k_out, v_out = pl.pallas_call(
        _copy_kernel,
        grid=(b // blk,),
        in_specs=[spec, spec],
        out_specs=[spec, spec],
        out_shape=[
            jax.ShapeDtypeStruct((b, s, h, d), k_val.dtype),
            jax.ShapeDtypeStruct((b, s, h, d), v_val.dtype),
        ],
    )(k_val, v_val)
    return (k_out, v_out)
